# baseline (speedup 1.0000x reference)
import functools

import jax
import jax.numpy as jnp
import numpy as np
from jax import lax
from jax.experimental import pallas as pl
from jax.experimental.pallas import tpu as pltpu

_N = 1024
_K = 7
_BETA = 0.75
_NB = int(_BETA * _N)


def _grid_sample(img, coords):
    B, C, H, W = img.shape
    x = ((coords[..., 0] + 1.0) * W - 1.0) / 2.0
    y = ((coords[..., 1] + 1.0) * H - 1.0) / 2.0
    x0 = jnp.floor(x); y0 = jnp.floor(y)
    x1 = x0 + 1.0; y1 = y0 + 1.0
    wa = (x1 - x) * (y1 - y)
    wb = (x1 - x) * (y - y0)
    wc = (x - x0) * (y1 - y)
    wd = (x - x0) * (y - y0)
    flat = img.reshape(B, C, H * W)
    N = coords.shape[1]

    def gather(ix, iy):
        valid = ((ix >= 0) & (ix <= W - 1) & (iy >= 0) & (iy <= H - 1)).astype(img.dtype)
        ixc = jnp.clip(ix, 0, W - 1).astype(jnp.int32)
        iyc = jnp.clip(iy, 0, H - 1).astype(jnp.int32)
        idx = iyc * W + ixc
        vals = jnp.take_along_axis(flat, jnp.broadcast_to(idx[:, None, :], (B, C, N)), axis=2)
        return vals * valid[:, None, :]

    out = (gather(x0, y0) * wa[:, None, :] + gather(x0, y1) * wb[:, None, :]
           + gather(x1, y0) * wc[:, None, :] + gather(x1, y1) * wd[:, None, :])
    return out


def _point_sample(img, points):
    return _grid_sample(img, 2.0 * points - 1.0)


def _sampling_points(mask, key):
    B = mask.shape[0]
    mask_sorted = -jnp.sort(-mask, axis=1)
    k1, k2 = jax.random.split(key)
    over = jax.random.uniform(k1, (B, _K * _N, 2), dtype=mask.dtype)
    over_map = _point_sample(mask_sorted, over)
    unc = -1.0 * (over_map[:, 0] - over_map[:, 1])
    _, idx = jax.lax.top_k(unc, _NB)
    importance = jnp.take_along_axis(over, idx[:, :, None], axis=1)
    coverage = jax.random.uniform(k2, (B, _N - _NB, 2), dtype=mask.dtype)
    return jnp.concatenate([importance, coverage], axis=1)


def _mlp_body(feat_ref, w1_ref, b1_ref, w2_ref, b2_ref, out_ref):
    feat = feat_ref[0]
    w1 = w1_ref[...]
    h = lax.dot_general(w1, feat, (((0,), (0,)), ((), ())),
                        preferred_element_type=jnp.float32)
    h = jnp.maximum(h + b1_ref[...].reshape(256, 1), 0.0)
    w2 = w2_ref[...]
    r = lax.dot_general(w2, h, (((0,), (0,)), ((), ())),
                        preferred_element_type=jnp.float32)
    out_ref[0] = r + b2_ref[...].reshape(8, 1)


def _mlp(feat, W1, b1, W2, b2):
    B, C, N = feat.shape
    Cp = 136
    featp = jnp.pad(feat, ((0, 0), (0, Cp - C), (0, 0)))
    W1p = jnp.pad(W1, ((0, Cp - C), (0, 0)))
    W2p = jnp.pad(W2, ((0, 0), (0, 1)))
    b2p = jnp.pad(b2, (0, 1))
    out = pl.pallas_call(
        _mlp_body,
        grid=(B,),
        in_specs=[
            pl.BlockSpec((1, Cp, N), lambda b: (b, 0, 0)),
            pl.BlockSpec((Cp, 256), lambda b: (0, 0)),
            pl.BlockSpec((256,), lambda b: (0,)),
            pl.BlockSpec((256, 8), lambda b: (0, 0)),
            pl.BlockSpec((8,), lambda b: (0,)),
        ],
        out_specs=pl.BlockSpec((1, 8, N), lambda b: (b, 0, 0)),
        out_shape=jax.ShapeDtypeStruct((B, 8, N), jnp.float32),
    )(featp, W1p, b1, W2p, b2p)
    return out[:, :7, :]


def kernel(fine, coarse, W1, b1, W2, b2):
    pkey = jax.random.key(42)
    points = _sampling_points(coarse, pkey)
    coarse_s = _point_sample(coarse, points)
    fine_s = _point_sample(fine, points)
    feat = jnp.concatenate([coarse_s, fine_s], axis=1)
    rend = _mlp(feat, W1, b1, W2, b2)
    return rend, points


# speedup vs baseline: 2.5615x; 2.5615x over previous
import functools

import jax
import jax.numpy as jnp
import numpy as np
from jax import lax
from jax.experimental import pallas as pl
from jax.experimental.pallas import tpu as pltpu
from jax.experimental.pallas import tpu_sc as plsc

_N = 1024
_K = 7
_NB = 768
_NOVER = _K * _N
_B = 4
_CH_C = 7
_CH_F = 128
_HC = 128
_HF = 256
_NTPB = 8


def _floor_f32(x):
    xi = x.astype(jnp.int32)
    xf = xi.astype(jnp.float32)
    return jnp.where(x < xf, xf - 1.0, xf)


def _corner_data(px, py, wh):
    whf = float(wh)
    cx = 2.0 * px - 1.0
    cy = 2.0 * py - 1.0
    x = ((cx + 1.0) * whf - 1.0) / 2.0
    y = ((cy + 1.0) * whf - 1.0) / 2.0
    x0 = _floor_f32(x)
    y0 = _floor_f32(y)
    x1 = x0 + 1.0
    y1 = y0 + 1.0
    wa = (x1 - x) * (y1 - y)
    wb = (x1 - x) * (y - y0)
    wc = (x - x0) * (y1 - y)
    wd = (x - x0) * (y - y0)
    lim = whf - 1.0
    vx0 = (x0 >= 0.0) & (x0 <= lim)
    vx1 = (x1 >= 0.0) & (x1 <= lim)
    vy0 = (y0 >= 0.0) & (y0 <= lim)
    vy1 = (y1 >= 0.0) & (y1 <= lim)
    zero = jnp.zeros_like(x)
    one = jnp.ones_like(x)
    va = jnp.where(vx0 & vy0, one, zero)
    vb = jnp.where(vx0 & vy1, one, zero)
    vc = jnp.where(vx1 & vy0, one, zero)
    vd = jnp.where(vx1 & vy1, one, zero)
    xi0 = jnp.clip(x0, 0.0, lim).astype(jnp.int32)
    xi1 = jnp.clip(x1, 0.0, lim).astype(jnp.int32)
    yi0 = jnp.clip(y0, 0.0, lim).astype(jnp.int32)
    yi1 = jnp.clip(y1, 0.0, lim).astype(jnp.int32)
    pa = yi0 * wh + xi0
    pb = yi1 * wh + xi0
    pc = yi0 * wh + xi1
    pd = yi1 * wh + xi1
    return (pa, pb, pc, pd), (va * wa, vb * wb, vc * wc, vd * wd), (va, vb, vc, vd), (wa, wb, wc, wd)


def _unc_body(coarse_hbm, overx_hbm, overy_hbm, unc_hbm, cpl_v, ox_v, oy_v, out_v):
    wid = lax.axis_index("s") * 2 + lax.axis_index("c")
    b = wid // _NTPB
    s = wid % _NTPB
    npts = _NOVER // _NTPB
    pltpu.sync_copy(coarse_hbm.at[pl.ds(b * _CH_C * _HC * _HC, _CH_C * _HC * _HC)], cpl_v)
    pltpu.sync_copy(overx_hbm.at[pl.ds(b * _NOVER + s * npts, npts)], ox_v)
    pltpu.sync_copy(overy_hbm.at[pl.ds(b * _NOVER + s * npts, npts)], oy_v)

    def body(j, _):
        sl = pl.ds(j * 16, 16)
        px = ox_v[sl]
        py = oy_v[sl]
        pix, _, valid, w = _corner_data(px, py, _HC)
        ch0 = None
        ch1 = None
        for k in range(4):
            m1 = None
            m2 = None
            for c in range(_CH_C):
                cc = jnp.full((16,), c * _HC * _HC, jnp.int32)
                v = plsc.load_gather(cpl_v, [cc + pix[k]])
                if m1 is None:
                    m1 = v
                    m2 = jnp.full((16,), -np.inf, jnp.float32)
                else:
                    gt = v > m1
                    m2 = jnp.where(gt, m1, jnp.where(v > m2, v, m2))
                    m1 = jnp.where(gt, v, m1)
            t0 = (m1 * valid[k]) * w[k]
            t1 = (m2 * valid[k]) * w[k]
            ch0 = t0 if ch0 is None else ch0 + t0
            ch1 = t1 if ch1 is None else ch1 + t1
        out_v[sl] = -1.0 * (ch0 - ch1)
        return 0

    lax.fori_loop(0, npts // 16, body, 0)
    pltpu.sync_copy(out_v, unc_hbm.at[pl.ds(b * _NOVER + s * npts, npts)])


def _gather_body(fine_hbm, coarse_hbm, overx_hbm, overy_hbm, covx_hbm, covy_hbm,
                 idx_hbm, feat_hbm, px_hbm, py_hbm,
                 plane_v, ox_v, oy_v, idx_v, ptsx_v, ptsy_v,
                 fidx_v, fw_v, cidx_v, cw_v, out_v):
    wid = lax.axis_index("s") * 2 + lax.axis_index("c")
    b = wid // _NTPB
    s = wid % _NTPB

    pltpu.sync_copy(overx_hbm.at[pl.ds(b * _NOVER, _NOVER)], ox_v)
    pltpu.sync_copy(overy_hbm.at[pl.ds(b * _NOVER, _NOVER)], oy_v)
    pltpu.sync_copy(idx_hbm.at[pl.ds(b * _NB, _NB)], idx_v)
    pltpu.sync_copy(covx_hbm.at[pl.ds(b * (_N - _NB), _N - _NB)], ptsx_v.at[pl.ds(_NB, _N - _NB)])
    pltpu.sync_copy(covy_hbm.at[pl.ds(b * (_N - _NB), _N - _NB)], ptsy_v.at[pl.ds(_NB, _N - _NB)])

    def imp_body(j, _):
        sl = pl.ds(j * 16, 16)
        iv = idx_v[sl]
        ptsx_v[sl] = plsc.load_gather(ox_v, [iv])
        ptsy_v[sl] = plsc.load_gather(oy_v, [iv])
        return 0

    lax.fori_loop(0, _NB // 16, imp_body, 0)

    @pl.when(s == 0)
    def _():
        pltpu.sync_copy(ptsx_v, px_hbm.at[pl.ds(b * _N, _N)])
        pltpu.sync_copy(ptsy_v, py_hbm.at[pl.ds(b * _N, _N)])

    def cdata_body(j, _):
        sl = pl.ds(j * 16, 16)
        px = ptsx_v[sl]
        py = ptsy_v[sl]
        pixf, wvf, _, _ = _corner_data(px, py, _HF)
        for k in range(4):
            fidx_v[k, sl] = pixf[k]
            fw_v[k, sl] = wvf[k]
        pixc, wvc, _, _ = _corner_data(px, py, _HC)
        for k in range(4):
            cidx_v[k, sl] = pixc[k]
            cw_v[k, sl] = wvc[k]
        return 0

    lax.fori_loop(0, _N // 16, cdata_body, 0)

    def plane_body(c, _):
        plane = b * _CH_F + s * (_CH_F // _NTPB) + c
        pltpu.sync_copy(fine_hbm.at[pl.ds(plane * _HF * _HF, _HF * _HF)], plane_v)

        def g_body(j, _):
            sl = pl.ds(j * 16, 16)
            acc = None
            for k in range(4):
                v = plsc.load_gather(plane_v, [fidx_v[k, sl]])
                t = v * fw_v[k, sl]
                acc = t if acc is None else acc + t
            out_v[sl] = acc
            return 0

        lax.fori_loop(0, _N // 16, g_body, 0)
        row = b * 136 + _CH_C + s * (_CH_F // _NTPB) + c
        pltpu.sync_copy(out_v, feat_hbm.at[pl.ds(row * _N, _N)])
        return 0

    lax.fori_loop(0, _CH_F // _NTPB, plane_body, 0)

    @pl.when(s < _CH_C)
    def _():
        pltpu.sync_copy(coarse_hbm.at[pl.ds((b * _CH_C + s) * _HC * _HC, _HC * _HC)],
                        plane_v.at[pl.ds(0, _HC * _HC)])

        def cg_body(j, _):
            sl = pl.ds(j * 16, 16)
            acc = None
            for k in range(4):
                v = plsc.load_gather(plane_v.at[pl.ds(0, _HC * _HC)], [cidx_v[k, sl]])
                t = v * cw_v[k, sl]
                acc = t if acc is None else acc + t
            out_v[sl] = acc
            return 0

        lax.fori_loop(0, _N // 16, cg_body, 0)
        pltpu.sync_copy(out_v, feat_hbm.at[pl.ds((b * 136 + s) * _N, _N)])

    @pl.when(s == _CH_C)
    def _():
        def z_body(j, _):
            out_v[pl.ds(j * 16, 16)] = jnp.zeros((16,), jnp.float32)
            return 0

        lax.fori_loop(0, _N // 16, z_body, 0)
        pltpu.sync_copy(out_v, feat_hbm.at[pl.ds((b * 136 + 135) * _N, _N)])


def _sc_unc(coarse_flat, over_x, over_y):
    mesh = plsc.VectorSubcoreMesh(core_axis_name="c", subcore_axis_name="s")
    f = functools.partial(
        pl.kernel, _unc_body, mesh=mesh,
        compiler_params=pltpu.CompilerParams(needs_layout_passes=False),
        out_type=jax.ShapeDtypeStruct((_B * _NOVER,), jnp.float32),
        scratch_types=[
            pltpu.VMEM((_CH_C * _HC * _HC,), jnp.float32),
            pltpu.VMEM((_NOVER // _NTPB,), jnp.float32),
            pltpu.VMEM((_NOVER // _NTPB,), jnp.float32),
            pltpu.VMEM((_NOVER // _NTPB,), jnp.float32),
        ],
    )()
    return f(coarse_flat, over_x, over_y).reshape(_B, _NOVER)


def _sc_gather(fine_flat, coarse_flat, over_x, over_y, cov_x, cov_y, idx):
    mesh = plsc.VectorSubcoreMesh(core_axis_name="c", subcore_axis_name="s")
    f = functools.partial(
        pl.kernel, _gather_body, mesh=mesh,
        compiler_params=pltpu.CompilerParams(needs_layout_passes=False),
        out_type=(
            jax.ShapeDtypeStruct((_B * 136 * _N,), jnp.float32),
            jax.ShapeDtypeStruct((_B * _N,), jnp.float32),
            jax.ShapeDtypeStruct((_B * _N,), jnp.float32),
        ),
        scratch_types=[
            pltpu.VMEM((_HF * _HF,), jnp.float32),
            pltpu.VMEM((_NOVER,), jnp.float32),
            pltpu.VMEM((_NOVER,), jnp.float32),
            pltpu.VMEM((_NB,), jnp.int32),
            pltpu.VMEM((_N,), jnp.float32),
            pltpu.VMEM((_N,), jnp.float32),
            pltpu.VMEM((4, _N), jnp.int32),
            pltpu.VMEM((4, _N), jnp.float32),
            pltpu.VMEM((4, _N), jnp.int32),
            pltpu.VMEM((4, _N), jnp.float32),
            pltpu.VMEM((_N,), jnp.float32),
        ],
    )()
    featp, px, py = f(fine_flat, coarse_flat, over_x, over_y, cov_x, cov_y, idx)
    return (featp.reshape(_B, 136, _N), px.reshape(_B, _N), py.reshape(_B, _N))


def _mlp_body(feat_ref, w1_ref, b1_ref, w2_ref, b2_ref, out_ref):
    feat = feat_ref[0]
    w1 = w1_ref[...]
    h = lax.dot_general(w1, feat, (((0,), (0,)), ((), ())),
                        preferred_element_type=jnp.float32)
    h = jnp.maximum(h + b1_ref[...].reshape(256, 1), 0.0)
    w2 = w2_ref[...]
    r = lax.dot_general(w2, h, (((0,), (0,)), ((), ())),
                        preferred_element_type=jnp.float32)
    out_ref[0] = r + b2_ref[...].reshape(8, 1)


def _mlp(featp, W1, b1, W2, b2):
    B, Cp, N = featp.shape
    W1p = jnp.pad(W1, ((0, Cp - W1.shape[0]), (0, 0)))
    W2p = jnp.pad(W2, ((0, 0), (0, 1)))
    b2p = jnp.pad(b2, (0, 1))
    out = pl.pallas_call(
        _mlp_body,
        grid=(B,),
        in_specs=[
            pl.BlockSpec((1, Cp, N), lambda b: (b, 0, 0)),
            pl.BlockSpec((Cp, 256), lambda b: (0, 0)),
            pl.BlockSpec((256,), lambda b: (0,)),
            pl.BlockSpec((256, 8), lambda b: (0, 0)),
            pl.BlockSpec((8,), lambda b: (0,)),
        ],
        out_specs=pl.BlockSpec((1, 8, N), lambda b: (b, 0, 0)),
        out_shape=jax.ShapeDtypeStruct((B, 8, N), jnp.float32),
    )(featp, W1p, b1, W2p, b2p)
    return out[:, :7, :]


def kernel(fine, coarse, W1, b1, W2, b2):
    B = fine.shape[0]
    pkey = jax.random.key(42)
    k1, k2 = jax.random.split(pkey)
    over = jax.random.uniform(k1, (B, _NOVER, 2), dtype=coarse.dtype)
    coverage = jax.random.uniform(k2, (B, _N - _NB, 2), dtype=coarse.dtype)

    over_x = over[..., 0] + 0.0
    over_y = over[..., 1] + 0.0
    cov_x = coverage[..., 0] + 0.0
    cov_y = coverage[..., 1] + 0.0
    over_x = over_x.reshape(-1)
    over_y = over_y.reshape(-1)
    cov_x = cov_x.reshape(-1)
    cov_y = cov_y.reshape(-1)
    coarse_flat = coarse.reshape(-1)
    fine_flat = fine.reshape(-1)

    unc = _sc_unc(coarse_flat, over_x, over_y)
    _, idx = jax.lax.top_k(unc, _NB)
    idx = idx.reshape(-1)

    featp, px, py = _sc_gather(fine_flat, coarse_flat, over_x, over_y,
                               cov_x, cov_y, idx)
    rend = _mlp(featp, W1, b1, W2, b2)
    points = jnp.stack([px, py], axis=-1)
    return rend, points


# speedup vs baseline: 3.4617x; 1.3514x over previous
import functools

import jax
import jax.numpy as jnp
import numpy as np
from jax import lax
from jax.experimental import pallas as pl
from jax.experimental.pallas import tpu as pltpu
from jax.experimental.pallas import tpu_sc as plsc

_N = 1024
_K = 7
_NB = 768
_NOVER = _K * _N
_B = 4
_CH_C = 7
_CH_F = 128
_HC = 128
_HF = 256
_NTPB = 8


def _floor_f32(x):
    xi = x.astype(jnp.int32)
    xf = xi.astype(jnp.float32)
    return jnp.where(x < xf, xf - 1.0, xf)


def _corner_data(px, py, wh):
    whf = float(wh)
    cx = 2.0 * px - 1.0
    cy = 2.0 * py - 1.0
    x = ((cx + 1.0) * whf - 1.0) / 2.0
    y = ((cy + 1.0) * whf - 1.0) / 2.0
    x0 = _floor_f32(x)
    y0 = _floor_f32(y)
    x1 = x0 + 1.0
    y1 = y0 + 1.0
    wa = (x1 - x) * (y1 - y)
    wb = (x1 - x) * (y - y0)
    wc = (x - x0) * (y1 - y)
    wd = (x - x0) * (y - y0)
    lim = whf - 1.0
    vx0 = (x0 >= 0.0) & (x0 <= lim)
    vx1 = (x1 >= 0.0) & (x1 <= lim)
    vy0 = (y0 >= 0.0) & (y0 <= lim)
    vy1 = (y1 >= 0.0) & (y1 <= lim)
    zero = jnp.zeros_like(x)
    one = jnp.ones_like(x)
    va = jnp.where(vx0 & vy0, one, zero)
    vb = jnp.where(vx0 & vy1, one, zero)
    vc = jnp.where(vx1 & vy0, one, zero)
    vd = jnp.where(vx1 & vy1, one, zero)
    xi0 = jnp.clip(x0, 0.0, lim).astype(jnp.int32)
    xi1 = jnp.clip(x1, 0.0, lim).astype(jnp.int32)
    yi0 = jnp.clip(y0, 0.0, lim).astype(jnp.int32)
    yi1 = jnp.clip(y1, 0.0, lim).astype(jnp.int32)
    ys = (yi0, yi1, yi0, yi1)
    xs = (xi0, xi0, xi1, xi1)
    return ys, xs, (va * wa, vb * wb, vc * wc, vd * wd), (va, vb, vc, vd), (wa, wb, wc, wd)


def _unc_body(coarse_hbm, overx_hbm, overy_hbm, unc_hbm, cpl_v, ox_v, oy_v, out_v):
    wid = lax.axis_index("s") * 2 + lax.axis_index("c")
    b = wid // _NTPB
    s = wid % _NTPB
    npts = _NOVER // _NTPB
    pltpu.sync_copy(coarse_hbm.at[pl.ds(b * _CH_C * _HC, _CH_C * _HC)], cpl_v)
    pltpu.sync_copy(overx_hbm.at[pl.ds(b * _NOVER + s * npts, npts)], ox_v)
    pltpu.sync_copy(overy_hbm.at[pl.ds(b * _NOVER + s * npts, npts)], oy_v)

    def body(j, _):
        sl = pl.ds(j * 16, 16)
        px = ox_v[sl]
        py = oy_v[sl]
        ys, xs, _, valid, w = _corner_data(px, py, _HC)
        ch0 = None
        ch1 = None
        for k in range(4):
            m1 = None
            m2 = None
            for c in range(_CH_C):
                v = plsc.load_gather(cpl_v, [ys[k] + c * _HC, xs[k]])
                if m1 is None:
                    m1 = v
                    m2 = jnp.full((16,), -np.inf, jnp.float32)
                else:
                    gt = v > m1
                    m2 = jnp.where(gt, m1, jnp.where(v > m2, v, m2))
                    m1 = jnp.where(gt, v, m1)
            t0 = (m1 * valid[k]) * w[k]
            t1 = (m2 * valid[k]) * w[k]
            ch0 = t0 if ch0 is None else ch0 + t0
            ch1 = t1 if ch1 is None else ch1 + t1
        out_v[sl] = -1.0 * (ch0 - ch1)
        return 0

    lax.fori_loop(0, npts // 16, body, 0)
    pltpu.sync_copy(out_v, unc_hbm.at[pl.ds(b * _NOVER + s * npts, npts)])


def _gather_body(fine_hbm, coarse_hbm, overx_hbm, overy_hbm, covx_hbm, covy_hbm,
                 idx_hbm, feat_hbm, px_hbm, py_hbm,
                 plane_v, cplane_v, ox_v, oy_v, idx_v, ptsx_v, ptsy_v,
                 fidx_v, fw_v, cidx_v, cw_v, out_v):
    wid = lax.axis_index("s") * 2 + lax.axis_index("c")
    b = wid // _NTPB
    s = wid % _NTPB

    pltpu.sync_copy(overx_hbm.at[pl.ds(b * _NOVER, _NOVER)], ox_v)
    pltpu.sync_copy(overy_hbm.at[pl.ds(b * _NOVER, _NOVER)], oy_v)
    pltpu.sync_copy(idx_hbm.at[pl.ds(b * _NB, _NB)], idx_v)
    pltpu.sync_copy(covx_hbm.at[pl.ds(b * (_N - _NB), _N - _NB)], ptsx_v.at[pl.ds(_NB, _N - _NB)])
    pltpu.sync_copy(covy_hbm.at[pl.ds(b * (_N - _NB), _N - _NB)], ptsy_v.at[pl.ds(_NB, _N - _NB)])

    def imp_body(j, _):
        sl = pl.ds(j * 16, 16)
        iv = idx_v[sl]
        ptsx_v[sl] = plsc.load_gather(ox_v, [iv])
        ptsy_v[sl] = plsc.load_gather(oy_v, [iv])
        return 0

    lax.fori_loop(0, _NB // 16, imp_body, 0)

    @pl.when(s == 0)
    def _():
        pltpu.sync_copy(ptsx_v, px_hbm.at[pl.ds(b * _N, _N)])
        pltpu.sync_copy(ptsy_v, py_hbm.at[pl.ds(b * _N, _N)])

    def cdata_body(j, _):
        sl = pl.ds(j * 16, 16)
        px = ptsx_v[sl]
        py = ptsy_v[sl]
        fys, fxs, wvf, _, _ = _corner_data(px, py, _HF)
        for k in range(4):
            fidx_v[k, sl] = fys[k] * _HF + fxs[k]
            fw_v[k, sl] = wvf[k]
        cys, cxs, wvc, _, _ = _corner_data(px, py, _HC)
        for k in range(4):
            cidx_v[k, sl] = cys[k] * _HC + cxs[k]
            cw_v[k, sl] = wvc[k]
        return 0

    lax.fori_loop(0, _N // 16, cdata_body, 0)

    def plane_body(c, _):
        plane = b * _CH_F + s * (_CH_F // _NTPB) + c
        pltpu.sync_copy(fine_hbm.at[pl.ds(plane * _HF, _HF)], plane_v)

        def g_body(j, _):
            sl = pl.ds(j * 16, 16)
            acc = None
            for k in range(4):
                fi = fidx_v[k, sl]
                v = plsc.load_gather(plane_v, [fi >> 8, fi & 255])
                t = v * fw_v[k, sl]
                acc = t if acc is None else acc + t
            out_v[sl] = acc
            return 0

        lax.fori_loop(0, _N // 16, g_body, 0)
        row = b * 136 + _CH_C + s * (_CH_F // _NTPB) + c
        pltpu.sync_copy(out_v, feat_hbm.at[pl.ds(row * _N, _N)])
        return 0

    lax.fori_loop(0, _CH_F // _NTPB, plane_body, 0)

    @pl.when(s < _CH_C)
    def _():
        pltpu.sync_copy(coarse_hbm.at[pl.ds((b * _CH_C + s) * _HC, _HC)], cplane_v)

        def cg_body(j, _):
            sl = pl.ds(j * 16, 16)
            acc = None
            for k in range(4):
                ci = cidx_v[k, sl]
                v = plsc.load_gather(cplane_v, [ci >> 7, ci & 127])
                t = v * cw_v[k, sl]
                acc = t if acc is None else acc + t
            out_v[sl] = acc
            return 0

        lax.fori_loop(0, _N // 16, cg_body, 0)
        pltpu.sync_copy(out_v, feat_hbm.at[pl.ds((b * 136 + s) * _N, _N)])

    @pl.when(s == _CH_C)
    def _():
        def z_body(j, _):
            out_v[pl.ds(j * 16, 16)] = jnp.zeros((16,), jnp.float32)
            return 0

        lax.fori_loop(0, _N // 16, z_body, 0)
        pltpu.sync_copy(out_v, feat_hbm.at[pl.ds((b * 136 + 135) * _N, _N)])


def _sc_unc(coarse_flat, over_x, over_y):
    mesh = plsc.VectorSubcoreMesh(core_axis_name="c", subcore_axis_name="s")
    f = functools.partial(
        pl.kernel, _unc_body, mesh=mesh,
        compiler_params=pltpu.CompilerParams(needs_layout_passes=False),
        out_type=jax.ShapeDtypeStruct((_B * _NOVER,), jnp.float32),
        scratch_types=[
            pltpu.VMEM((_CH_C * _HC, _HC), jnp.float32),
            pltpu.VMEM((_NOVER // _NTPB,), jnp.float32),
            pltpu.VMEM((_NOVER // _NTPB,), jnp.float32),
            pltpu.VMEM((_NOVER // _NTPB,), jnp.float32),
        ],
    )()
    return f(coarse_flat, over_x, over_y).reshape(_B, _NOVER)


def _sc_gather(fine_flat, coarse_flat, over_x, over_y, cov_x, cov_y, idx):
    mesh = plsc.VectorSubcoreMesh(core_axis_name="c", subcore_axis_name="s")
    f = functools.partial(
        pl.kernel, _gather_body, mesh=mesh,
        compiler_params=pltpu.CompilerParams(needs_layout_passes=False),
        out_type=(
            jax.ShapeDtypeStruct((_B * 136 * _N,), jnp.float32),
            jax.ShapeDtypeStruct((_B * _N,), jnp.float32),
            jax.ShapeDtypeStruct((_B * _N,), jnp.float32),
        ),
        scratch_types=[
            pltpu.VMEM((_HF, _HF), jnp.float32),
            pltpu.VMEM((_HC, _HC), jnp.float32),
            pltpu.VMEM((_NOVER,), jnp.float32),
            pltpu.VMEM((_NOVER,), jnp.float32),
            pltpu.VMEM((_NB,), jnp.int32),
            pltpu.VMEM((_N,), jnp.float32),
            pltpu.VMEM((_N,), jnp.float32),
            pltpu.VMEM((4, _N), jnp.int32),
            pltpu.VMEM((4, _N), jnp.float32),
            pltpu.VMEM((4, _N), jnp.int32),
            pltpu.VMEM((4, _N), jnp.float32),
            pltpu.VMEM((_N,), jnp.float32),
        ],
    )()
    featp, px, py = f(fine_flat, coarse_flat, over_x, over_y, cov_x, cov_y, idx)
    return (featp.reshape(_B, 136, _N), px.reshape(_B, _N), py.reshape(_B, _N))


def _mlp_body(feat_ref, w1_ref, b1_ref, w2_ref, b2_ref, out_ref):
    feat = feat_ref[0]
    w1 = w1_ref[...]
    h = lax.dot_general(w1, feat, (((0,), (0,)), ((), ())),
                        preferred_element_type=jnp.float32)
    h = jnp.maximum(h + b1_ref[...].reshape(256, 1), 0.0)
    w2 = w2_ref[...]
    r = lax.dot_general(w2, h, (((0,), (0,)), ((), ())),
                        preferred_element_type=jnp.float32)
    out_ref[0] = r + b2_ref[...].reshape(8, 1)


def _mlp(featp, W1, b1, W2, b2):
    B, Cp, N = featp.shape
    W1p = jnp.pad(W1, ((0, Cp - W1.shape[0]), (0, 0)))
    W2p = jnp.pad(W2, ((0, 0), (0, 1)))
    b2p = jnp.pad(b2, (0, 1))
    out = pl.pallas_call(
        _mlp_body,
        grid=(B,),
        in_specs=[
            pl.BlockSpec((1, Cp, N), lambda b: (b, 0, 0)),
            pl.BlockSpec((Cp, 256), lambda b: (0, 0)),
            pl.BlockSpec((256,), lambda b: (0,)),
            pl.BlockSpec((256, 8), lambda b: (0, 0)),
            pl.BlockSpec((8,), lambda b: (0,)),
        ],
        out_specs=pl.BlockSpec((1, 8, N), lambda b: (b, 0, 0)),
        out_shape=jax.ShapeDtypeStruct((B, 8, N), jnp.float32),
    )(featp, W1p, b1, W2p, b2p)
    return out[:, :7, :]


def kernel(fine, coarse, W1, b1, W2, b2):
    B = fine.shape[0]
    pkey = jax.random.key(42)
    k1, k2 = jax.random.split(pkey)
    over = jax.random.uniform(k1, (B, _NOVER, 2), dtype=coarse.dtype)
    coverage = jax.random.uniform(k2, (B, _N - _NB, 2), dtype=coarse.dtype)

    over_x = over[..., 0] + 0.0
    over_y = over[..., 1] + 0.0
    cov_x = coverage[..., 0] + 0.0
    cov_y = coverage[..., 1] + 0.0
    over_x = over_x.reshape(-1)
    over_y = over_y.reshape(-1)
    cov_x = cov_x.reshape(-1)
    cov_y = cov_y.reshape(-1)
    coarse_flat = coarse.reshape(_B * _CH_C * _HC, _HC)
    fine_flat = fine.reshape(_B * _CH_F * _HF, _HF)

    unc = _sc_unc(coarse_flat, over_x, over_y)
    _, idx = jax.lax.top_k(unc, _NB)
    idx = idx.reshape(-1)

    featp, px, py = _sc_gather(fine_flat, coarse_flat, over_x, over_y,
                               cov_x, cov_y, idx)
    rend = _mlp(featp, W1, b1, W2, b2)
    points = jnp.stack([px, py], axis=-1)
    return rend, points


# speedup vs baseline: 3.8021x; 1.0983x over previous
import functools

import jax
import jax.numpy as jnp
import numpy as np
from jax import lax
from jax.experimental import pallas as pl
from jax.experimental.pallas import tpu as pltpu
from jax.experimental.pallas import tpu_sc as plsc

_N = 1024
_K = 7
_NB = 768
_NOVER = _K * _N
_B = 4
_CH_C = 7
_CH_F = 128
_HC = 128
_HF = 256
_NTPB = 8


def _floor_f32(x):
    xi = x.astype(jnp.int32)
    xf = xi.astype(jnp.float32)
    return jnp.where(x < xf, xf - 1.0, xf)


def _corner_data(px, py, wh):
    whf = float(wh)
    cx = 2.0 * px - 1.0
    cy = 2.0 * py - 1.0
    x = ((cx + 1.0) * whf - 1.0) / 2.0
    y = ((cy + 1.0) * whf - 1.0) / 2.0
    x0 = _floor_f32(x)
    y0 = _floor_f32(y)
    x1 = x0 + 1.0
    y1 = y0 + 1.0
    wa = (x1 - x) * (y1 - y)
    wb = (x1 - x) * (y - y0)
    wc = (x - x0) * (y1 - y)
    wd = (x - x0) * (y - y0)
    lim = whf - 1.0
    vx0 = (x0 >= 0.0) & (x0 <= lim)
    vx1 = (x1 >= 0.0) & (x1 <= lim)
    vy0 = (y0 >= 0.0) & (y0 <= lim)
    vy1 = (y1 >= 0.0) & (y1 <= lim)
    zero = jnp.zeros_like(x)
    one = jnp.ones_like(x)
    va = jnp.where(vx0 & vy0, one, zero)
    vb = jnp.where(vx0 & vy1, one, zero)
    vc = jnp.where(vx1 & vy0, one, zero)
    vd = jnp.where(vx1 & vy1, one, zero)
    xi0 = jnp.clip(x0, 0.0, lim).astype(jnp.int32)
    xi1 = jnp.clip(x1, 0.0, lim).astype(jnp.int32)
    yi0 = jnp.clip(y0, 0.0, lim).astype(jnp.int32)
    yi1 = jnp.clip(y1, 0.0, lim).astype(jnp.int32)
    ys = (yi0, yi1, yi0, yi1)
    xs = (xi0, xi0, xi1, xi1)
    return ys, xs, (va * wa, vb * wb, vc * wc, vd * wd), (va, vb, vc, vd), (wa, wb, wc, wd)


def _unc_body(coarse_hbm, overx_hbm, overy_hbm, unc_hbm, cpl_v, ox_v, oy_v, out_v):
    wid = lax.axis_index("s") * 2 + lax.axis_index("c")
    b = wid // _NTPB
    s = wid % _NTPB
    npts = _NOVER // _NTPB
    pltpu.sync_copy(coarse_hbm.at[pl.ds(b * _CH_C * _HC, _CH_C * _HC)], cpl_v)
    pltpu.sync_copy(overx_hbm.at[pl.ds(b * _NOVER + s * npts, npts)], ox_v)
    pltpu.sync_copy(overy_hbm.at[pl.ds(b * _NOVER + s * npts, npts)], oy_v)

    def body(j, _):
        sl = pl.ds(j * 16, 16)
        px = ox_v[sl]
        py = oy_v[sl]
        ys, xs, _, valid, w = _corner_data(px, py, _HC)
        ch0 = None
        ch1 = None
        for k in range(4):
            m1 = None
            m2 = None
            for c in range(_CH_C):
                v = plsc.load_gather(cpl_v, [ys[k] + c * _HC, xs[k]])
                if m1 is None:
                    m1 = v
                    m2 = jnp.full((16,), -np.inf, jnp.float32)
                else:
                    gt = v > m1
                    m2 = jnp.where(gt, m1, jnp.where(v > m2, v, m2))
                    m1 = jnp.where(gt, v, m1)
            t0 = (m1 * valid[k]) * w[k]
            t1 = (m2 * valid[k]) * w[k]
            ch0 = t0 if ch0 is None else ch0 + t0
            ch1 = t1 if ch1 is None else ch1 + t1
        out_v[sl] = -1.0 * (ch0 - ch1)
        return 0

    lax.fori_loop(0, npts // 16, body, 0)
    pltpu.sync_copy(out_v, unc_hbm.at[pl.ds(b * _NOVER + s * npts, npts)])


def _gather_body(fine_hbm, coarse_hbm, overx_hbm, overy_hbm, covx_hbm, covy_hbm,
                 idx_hbm, feat_hbm, px_hbm, py_hbm,
                 plane_v, cplane_v, ox_v, oy_v, idx_v, ptsx_v, ptsy_v,
                 fidx_v, fw_v, cidx_v, cw_v, out_v, out2_v, sem0, sem1, semo):
    wid = lax.axis_index("s") * 2 + lax.axis_index("c")
    b = wid // _NTPB
    s = wid % _NTPB

    pltpu.sync_copy(overx_hbm.at[pl.ds(b * _NOVER, _NOVER)], ox_v)
    pltpu.sync_copy(overy_hbm.at[pl.ds(b * _NOVER, _NOVER)], oy_v)
    pltpu.sync_copy(idx_hbm.at[pl.ds(b * _NB, _NB)], idx_v)
    pltpu.sync_copy(covx_hbm.at[pl.ds(b * (_N - _NB), _N - _NB)], ptsx_v.at[pl.ds(_NB, _N - _NB)])
    pltpu.sync_copy(covy_hbm.at[pl.ds(b * (_N - _NB), _N - _NB)], ptsy_v.at[pl.ds(_NB, _N - _NB)])

    def imp_body(j, _):
        sl = pl.ds(j * 16, 16)
        iv = idx_v[sl]
        ptsx_v[sl] = plsc.load_gather(ox_v, [iv])
        ptsy_v[sl] = plsc.load_gather(oy_v, [iv])
        return 0

    lax.fori_loop(0, _NB // 16, imp_body, 0)

    @pl.when(s == 0)
    def _():
        pltpu.sync_copy(ptsx_v, px_hbm.at[pl.ds(b * _N, _N)])
        pltpu.sync_copy(ptsy_v, py_hbm.at[pl.ds(b * _N, _N)])

    def cdata_body(j, _):
        sl = pl.ds(j * 16, 16)
        px = ptsx_v[sl]
        py = ptsy_v[sl]
        fys, fxs, wvf, _, _ = _corner_data(px, py, _HF)
        for k in range(4):
            fidx_v[k, sl] = fys[k] * _HF + fxs[k]
            fw_v[k, sl] = wvf[k]
        cys, cxs, wvc, _, _ = _corner_data(px, py, _HC)
        for k in range(4):
            cidx_v[k, sl] = cys[k] * _HC + cxs[k]
            cw_v[k, sl] = wvc[k]
        return 0

    lax.fori_loop(0, _N // 16, cdata_body, 0)

    def start_half(p, h):
        plane = b * _CH_F + s * (_CH_F // _NTPB) + p
        return pltpu.async_copy(
            fine_hbm.at[pl.ds((plane * 2 + h) * (_HF // 2), _HF // 2)],
            plane_v.at[pl.ds(h * (_HF // 2), _HF // 2)],
            sem0 if h == 0 else sem1)

    def g_pass(h, first, ov):
        def g_body(j, _):
            sl = pl.ds(j * 16, 16)
            acc = None
            for k in range(4):
                fi = fidx_v[k, sl]
                v = plsc.load_gather(plane_v, [fi >> 8, fi & 255])
                t = jnp.where((fi >> 15) == h, v * fw_v[k, sl],
                              jnp.zeros((16,), jnp.float32))
                acc = t if acc is None else acc + t
            ov[sl] = acc if first else ov[sl] + acc
            return 0

        lax.fori_loop(0, _N // 16, g_body, 0)

    h0 = start_half(0, 0)
    h1 = start_half(0, 1)
    out_handles = [None] * (_CH_F // _NTPB)
    for p in range(_CH_F // _NTPB):
        ov = out_v if p % 2 == 0 else out2_v
        if p >= 2 and out_handles[p - 2] is not None:
            out_handles[p - 2].wait()
        h0.wait()
        g_pass(0, True, ov)
        if p + 1 < _CH_F // _NTPB:
            h0 = start_half(p + 1, 0)
        h1.wait()
        g_pass(1, False, ov)
        if p + 1 < _CH_F // _NTPB:
            h1 = start_half(p + 1, 1)
        row = b * 136 + _CH_C + s * (_CH_F // _NTPB) + p
        out_handles[p] = pltpu.async_copy(ov, feat_hbm.at[pl.ds(row * _N, _N)], semo)
    out_handles[-2].wait()
    out_handles[-1].wait()

    @pl.when(s < _CH_C)
    def _():
        pltpu.sync_copy(coarse_hbm.at[pl.ds((b * _CH_C + s) * _HC, _HC)], cplane_v)

        def cg_body(j, _):
            sl = pl.ds(j * 16, 16)
            acc = None
            for k in range(4):
                ci = cidx_v[k, sl]
                v = plsc.load_gather(cplane_v, [ci >> 7, ci & 127])
                t = v * cw_v[k, sl]
                acc = t if acc is None else acc + t
            out_v[sl] = acc
            return 0

        lax.fori_loop(0, _N // 16, cg_body, 0)
        pltpu.sync_copy(out_v, feat_hbm.at[pl.ds((b * 136 + s) * _N, _N)])

    @pl.when(s == _CH_C)
    def _():
        def z_body(j, _):
            out_v[pl.ds(j * 16, 16)] = jnp.zeros((16,), jnp.float32)
            return 0

        lax.fori_loop(0, _N // 16, z_body, 0)
        pltpu.sync_copy(out_v, feat_hbm.at[pl.ds((b * 136 + 135) * _N, _N)])


def _sc_unc(coarse_flat, over_x, over_y):
    mesh = plsc.VectorSubcoreMesh(core_axis_name="c", subcore_axis_name="s")
    f = functools.partial(
        pl.kernel, _unc_body, mesh=mesh,
        compiler_params=pltpu.CompilerParams(needs_layout_passes=False),
        out_type=jax.ShapeDtypeStruct((_B * _NOVER,), jnp.float32),
        scratch_types=[
            pltpu.VMEM((_CH_C * _HC, _HC), jnp.float32),
            pltpu.VMEM((_NOVER // _NTPB,), jnp.float32),
            pltpu.VMEM((_NOVER // _NTPB,), jnp.float32),
            pltpu.VMEM((_NOVER // _NTPB,), jnp.float32),
        ],
    )()
    return f(coarse_flat, over_x, over_y).reshape(_B, _NOVER)


def _sc_gather(fine_flat, coarse_flat, over_x, over_y, cov_x, cov_y, idx):
    mesh = plsc.VectorSubcoreMesh(core_axis_name="c", subcore_axis_name="s")
    f = functools.partial(
        pl.kernel, _gather_body, mesh=mesh,
        compiler_params=pltpu.CompilerParams(needs_layout_passes=False),
        out_type=(
            jax.ShapeDtypeStruct((_B * 136 * _N,), jnp.float32),
            jax.ShapeDtypeStruct((_B * _N,), jnp.float32),
            jax.ShapeDtypeStruct((_B * _N,), jnp.float32),
        ),
        scratch_types=[
            pltpu.VMEM((_HF, _HF), jnp.float32),
            pltpu.VMEM((_HC, _HC), jnp.float32),
            pltpu.VMEM((_NOVER,), jnp.float32),
            pltpu.VMEM((_NOVER,), jnp.float32),
            pltpu.VMEM((_NB,), jnp.int32),
            pltpu.VMEM((_N,), jnp.float32),
            pltpu.VMEM((_N,), jnp.float32),
            pltpu.VMEM((4, _N), jnp.int32),
            pltpu.VMEM((4, _N), jnp.float32),
            pltpu.VMEM((4, _N), jnp.int32),
            pltpu.VMEM((4, _N), jnp.float32),
            pltpu.VMEM((_N,), jnp.float32),
            pltpu.VMEM((_N,), jnp.float32),
            pltpu.SemaphoreType.DMA,
            pltpu.SemaphoreType.DMA,
            pltpu.SemaphoreType.DMA,
        ],
    )()
    featp, px, py = f(fine_flat, coarse_flat, over_x, over_y, cov_x, cov_y, idx)
    return (featp.reshape(_B, 136, _N), px.reshape(_B, _N), py.reshape(_B, _N))


def _mlp_body(feat_ref, w1_ref, b1_ref, w2_ref, b2_ref, out_ref):
    feat = feat_ref[0]
    w1 = w1_ref[...]
    h = lax.dot_general(w1, feat, (((0,), (0,)), ((), ())),
                        preferred_element_type=jnp.float32)
    h = jnp.maximum(h + b1_ref[...].reshape(256, 1), 0.0)
    w2 = w2_ref[...]
    r = lax.dot_general(w2, h, (((0,), (0,)), ((), ())),
                        preferred_element_type=jnp.float32)
    out_ref[0] = r + b2_ref[...].reshape(8, 1)


def _mlp(featp, W1, b1, W2, b2):
    B, Cp, N = featp.shape
    W1p = jnp.pad(W1, ((0, Cp - W1.shape[0]), (0, 0)))
    W2p = jnp.pad(W2, ((0, 0), (0, 1)))
    b2p = jnp.pad(b2, (0, 1))
    out = pl.pallas_call(
        _mlp_body,
        grid=(B,),
        in_specs=[
            pl.BlockSpec((1, Cp, N), lambda b: (b, 0, 0)),
            pl.BlockSpec((Cp, 256), lambda b: (0, 0)),
            pl.BlockSpec((256,), lambda b: (0,)),
            pl.BlockSpec((256, 8), lambda b: (0, 0)),
            pl.BlockSpec((8,), lambda b: (0,)),
        ],
        out_specs=pl.BlockSpec((1, 8, N), lambda b: (b, 0, 0)),
        out_shape=jax.ShapeDtypeStruct((B, 8, N), jnp.float32),
    )(featp, W1p, b1, W2p, b2p)
    return out[:, :7, :]


def kernel(fine, coarse, W1, b1, W2, b2):
    B = fine.shape[0]
    pkey = jax.random.key(42)
    k1, k2 = jax.random.split(pkey)
    over = jax.random.uniform(k1, (B, _NOVER, 2), dtype=coarse.dtype)
    coverage = jax.random.uniform(k2, (B, _N - _NB, 2), dtype=coarse.dtype)

    over_x = over[..., 0] + 0.0
    over_y = over[..., 1] + 0.0
    cov_x = coverage[..., 0] + 0.0
    cov_y = coverage[..., 1] + 0.0
    over_x = over_x.reshape(-1)
    over_y = over_y.reshape(-1)
    cov_x = cov_x.reshape(-1)
    cov_y = cov_y.reshape(-1)
    coarse_flat = coarse.reshape(_B * _CH_C * _HC, _HC)
    fine_flat = fine.reshape(_B * _CH_F * _HF, _HF)

    unc = _sc_unc(coarse_flat, over_x, over_y)
    _, idx = jax.lax.top_k(unc, _NB)
    idx = idx.reshape(-1)

    featp, px, py = _sc_gather(fine_flat, coarse_flat, over_x, over_y,
                               cov_x, cov_y, idx)
    rend = _mlp(featp, W1, b1, W2, b2)
    points = jnp.stack([px, py], axis=-1)
    return rend, points


# speedup vs baseline: 3.8042x; 1.0005x over previous
import functools

import jax
import jax.numpy as jnp
import numpy as np
from jax import lax
from jax.experimental import pallas as pl
from jax.experimental.pallas import tpu as pltpu
from jax.experimental.pallas import tpu_sc as plsc

_N = 1024
_K = 7
_NB = 768
_NOVER = _K * _N
_B = 4
_CH_C = 7
_CH_F = 128
_HC = 128
_HF = 256
_NTPB = 8


def _floor_f32(x):
    xi = x.astype(jnp.int32)
    xf = xi.astype(jnp.float32)
    return jnp.where(x < xf, xf - 1.0, xf)


def _corner_data(px, py, wh):
    whf = float(wh)
    cx = 2.0 * px - 1.0
    cy = 2.0 * py - 1.0
    x = ((cx + 1.0) * whf - 1.0) / 2.0
    y = ((cy + 1.0) * whf - 1.0) / 2.0
    x0 = _floor_f32(x)
    y0 = _floor_f32(y)
    x1 = x0 + 1.0
    y1 = y0 + 1.0
    wa = (x1 - x) * (y1 - y)
    wb = (x1 - x) * (y - y0)
    wc = (x - x0) * (y1 - y)
    wd = (x - x0) * (y - y0)
    lim = whf - 1.0
    vx0 = (x0 >= 0.0) & (x0 <= lim)
    vx1 = (x1 >= 0.0) & (x1 <= lim)
    vy0 = (y0 >= 0.0) & (y0 <= lim)
    vy1 = (y1 >= 0.0) & (y1 <= lim)
    zero = jnp.zeros_like(x)
    one = jnp.ones_like(x)
    va = jnp.where(vx0 & vy0, one, zero)
    vb = jnp.where(vx0 & vy1, one, zero)
    vc = jnp.where(vx1 & vy0, one, zero)
    vd = jnp.where(vx1 & vy1, one, zero)
    xi0 = jnp.clip(x0, 0.0, lim).astype(jnp.int32)
    xi1 = jnp.clip(x1, 0.0, lim).astype(jnp.int32)
    yi0 = jnp.clip(y0, 0.0, lim).astype(jnp.int32)
    yi1 = jnp.clip(y1, 0.0, lim).astype(jnp.int32)
    ys = (yi0, yi1, yi0, yi1)
    xs = (xi0, xi0, xi1, xi1)
    return ys, xs, (va * wa, vb * wb, vc * wc, vd * wd), (va, vb, vc, vd), (wa, wb, wc, wd)


def _unc_body(coarse_hbm, overx_hbm, overy_hbm, unc_hbm, cpl_v, ox_v, oy_v, out_v):
    wid = lax.axis_index("s") * 2 + lax.axis_index("c")
    b = wid // _NTPB
    s = wid % _NTPB
    npts = _NOVER // _NTPB
    pltpu.sync_copy(coarse_hbm.at[pl.ds(b * _CH_C * _HC, _CH_C * _HC)], cpl_v)
    pltpu.sync_copy(overx_hbm.at[pl.ds(b * _NOVER + s * npts, npts)], ox_v)
    pltpu.sync_copy(overy_hbm.at[pl.ds(b * _NOVER + s * npts, npts)], oy_v)

    def body(j, _):
        sl = pl.ds(j * 16, 16)
        px = ox_v[sl]
        py = oy_v[sl]
        ys, xs, _, valid, w = _corner_data(px, py, _HC)
        ch0 = None
        ch1 = None
        for k in range(4):
            m1 = None
            m2 = None
            for c in range(_CH_C):
                v = plsc.load_gather(cpl_v, [ys[k] + c * _HC, xs[k]])
                if m1 is None:
                    m1 = v
                    m2 = jnp.full((16,), -np.inf, jnp.float32)
                else:
                    gt = v > m1
                    m2 = jnp.where(gt, m1, jnp.where(v > m2, v, m2))
                    m1 = jnp.where(gt, v, m1)
            t0 = (m1 * valid[k]) * w[k]
            t1 = (m2 * valid[k]) * w[k]
            ch0 = t0 if ch0 is None else ch0 + t0
            ch1 = t1 if ch1 is None else ch1 + t1
        out_v[sl] = -1.0 * (ch0 - ch1)
        return 0

    lax.fori_loop(0, npts // 16, body, 0)
    pltpu.sync_copy(out_v, unc_hbm.at[pl.ds(b * _NOVER + s * npts, npts)])


def _gather_body(fine_hbm, coarse_hbm, overx_hbm, overy_hbm, covx_hbm, covy_hbm,
                 idx_hbm, feat_hbm, px_hbm, py_hbm,
                 plane_v, cplane_v, ox_v, oy_v, idx_v, ptsx_v, ptsy_v,
                 fidx_v, fw0_v, fw1_v, cidx_v, cw_v, out_v, out2_v, sem0, sem1, semo):
    wid = lax.axis_index("s") * 2 + lax.axis_index("c")
    b = wid // _NTPB
    s = wid % _NTPB

    pltpu.sync_copy(overx_hbm.at[pl.ds(b * _NOVER, _NOVER)], ox_v)
    pltpu.sync_copy(overy_hbm.at[pl.ds(b * _NOVER, _NOVER)], oy_v)
    pltpu.sync_copy(idx_hbm.at[pl.ds(b * _NB, _NB)], idx_v)
    pltpu.sync_copy(covx_hbm.at[pl.ds(b * (_N - _NB), _N - _NB)], ptsx_v.at[pl.ds(_NB, _N - _NB)])
    pltpu.sync_copy(covy_hbm.at[pl.ds(b * (_N - _NB), _N - _NB)], ptsy_v.at[pl.ds(_NB, _N - _NB)])

    def imp_body(j, _):
        sl = pl.ds(j * 16, 16)
        iv = idx_v[sl]
        ptsx_v[sl] = plsc.load_gather(ox_v, [iv])
        ptsy_v[sl] = plsc.load_gather(oy_v, [iv])
        return 0

    lax.fori_loop(0, _NB // 16, imp_body, 0)

    @pl.when(s == 0)
    def _():
        pltpu.sync_copy(ptsx_v, px_hbm.at[pl.ds(b * _N, _N)])
        pltpu.sync_copy(ptsy_v, py_hbm.at[pl.ds(b * _N, _N)])

    def cdata_body(j, _):
        sl = pl.ds(j * 16, 16)
        px = ptsx_v[sl]
        py = ptsy_v[sl]
        fys, fxs, wvf, _, _ = _corner_data(px, py, _HF)
        zero = jnp.zeros((16,), jnp.float32)
        for k in range(4):
            fidx_v[k, sl] = fys[k] * _HF + fxs[k]
            w0 = jnp.where(fys[k] < (_HF // 2), wvf[k], zero)
            fw0_v[k, sl] = w0
            fw1_v[k, sl] = wvf[k] - w0
        cys, cxs, wvc, _, _ = _corner_data(px, py, _HC)
        for k in range(4):
            cidx_v[k, sl] = cys[k] * _HC + cxs[k]
            cw_v[k, sl] = wvc[k]
        return 0

    lax.fori_loop(0, _N // 16, cdata_body, 0)

    def start_half(p, h):
        plane = b * _CH_F + s * (_CH_F // _NTPB) + p
        return pltpu.async_copy(
            fine_hbm.at[pl.ds((plane * 2 + h) * (_HF // 2), _HF // 2)],
            plane_v.at[pl.ds(h * (_HF // 2), _HF // 2)],
            sem0 if h == 0 else sem1)

    def g_pass(h, first, ov):
        fw_h = fw0_v if h == 0 else fw1_v

        def g_body(j, _):
            sl = pl.ds(j * 16, 16)
            acc = None
            for k in range(4):
                fi = fidx_v[k, sl]
                v = plsc.load_gather(plane_v, [fi >> 8, fi & 255])
                t = v * fw_h[k, sl]
                acc = t if acc is None else acc + t
            ov[sl] = acc if first else ov[sl] + acc
            return 0

        lax.fori_loop(0, _N // 16, g_body, 0)

    h0 = start_half(0, 0)
    h1 = start_half(0, 1)
    out_handles = [None] * (_CH_F // _NTPB)
    for p in range(_CH_F // _NTPB):
        ov = out_v if p % 2 == 0 else out2_v
        if p >= 2 and out_handles[p - 2] is not None:
            out_handles[p - 2].wait()
        h0.wait()
        g_pass(0, True, ov)
        if p + 1 < _CH_F // _NTPB:
            h0 = start_half(p + 1, 0)
        h1.wait()
        g_pass(1, False, ov)
        if p + 1 < _CH_F // _NTPB:
            h1 = start_half(p + 1, 1)
        row = b * 136 + _CH_C + s * (_CH_F // _NTPB) + p
        out_handles[p] = pltpu.async_copy(ov, feat_hbm.at[pl.ds(row * _N, _N)], semo)
    out_handles[-2].wait()
    out_handles[-1].wait()

    @pl.when(s < _CH_C)
    def _():
        pltpu.sync_copy(coarse_hbm.at[pl.ds((b * _CH_C + s) * _HC, _HC)], cplane_v)

        def cg_body(j, _):
            sl = pl.ds(j * 16, 16)
            acc = None
            for k in range(4):
                ci = cidx_v[k, sl]
                v = plsc.load_gather(cplane_v, [ci >> 7, ci & 127])
                t = v * cw_v[k, sl]
                acc = t if acc is None else acc + t
            out_v[sl] = acc
            return 0

        lax.fori_loop(0, _N // 16, cg_body, 0)
        pltpu.sync_copy(out_v, feat_hbm.at[pl.ds((b * 136 + s) * _N, _N)])

    @pl.when(s == _CH_C)
    def _():
        def z_body(j, _):
            out_v[pl.ds(j * 16, 16)] = jnp.zeros((16,), jnp.float32)
            return 0

        lax.fori_loop(0, _N // 16, z_body, 0)
        pltpu.sync_copy(out_v, feat_hbm.at[pl.ds((b * 136 + 135) * _N, _N)])


def _sc_unc(coarse_flat, over_x, over_y):
    mesh = plsc.VectorSubcoreMesh(core_axis_name="c", subcore_axis_name="s")
    f = functools.partial(
        pl.kernel, _unc_body, mesh=mesh,
        compiler_params=pltpu.CompilerParams(needs_layout_passes=False),
        out_type=jax.ShapeDtypeStruct((_B * _NOVER,), jnp.float32),
        scratch_types=[
            pltpu.VMEM((_CH_C * _HC, _HC), jnp.float32),
            pltpu.VMEM((_NOVER // _NTPB,), jnp.float32),
            pltpu.VMEM((_NOVER // _NTPB,), jnp.float32),
            pltpu.VMEM((_NOVER // _NTPB,), jnp.float32),
        ],
    )()
    return f(coarse_flat, over_x, over_y).reshape(_B, _NOVER)


def _sc_gather(fine_flat, coarse_flat, over_x, over_y, cov_x, cov_y, idx):
    mesh = plsc.VectorSubcoreMesh(core_axis_name="c", subcore_axis_name="s")
    f = functools.partial(
        pl.kernel, _gather_body, mesh=mesh,
        compiler_params=pltpu.CompilerParams(needs_layout_passes=False),
        out_type=(
            jax.ShapeDtypeStruct((_B * 136 * _N,), jnp.float32),
            jax.ShapeDtypeStruct((_B * _N,), jnp.float32),
            jax.ShapeDtypeStruct((_B * _N,), jnp.float32),
        ),
        scratch_types=[
            pltpu.VMEM((_HF, _HF), jnp.float32),
            pltpu.VMEM((_HC, _HC), jnp.float32),
            pltpu.VMEM((_NOVER,), jnp.float32),
            pltpu.VMEM((_NOVER,), jnp.float32),
            pltpu.VMEM((_NB,), jnp.int32),
            pltpu.VMEM((_N,), jnp.float32),
            pltpu.VMEM((_N,), jnp.float32),
            pltpu.VMEM((4, _N), jnp.int32),
            pltpu.VMEM((4, _N), jnp.float32),
            pltpu.VMEM((4, _N), jnp.float32),
            pltpu.VMEM((4, _N), jnp.int32),
            pltpu.VMEM((4, _N), jnp.float32),
            pltpu.VMEM((_N,), jnp.float32),
            pltpu.VMEM((_N,), jnp.float32),
            pltpu.SemaphoreType.DMA,
            pltpu.SemaphoreType.DMA,
            pltpu.SemaphoreType.DMA,
        ],
    )()
    featp, px, py = f(fine_flat, coarse_flat, over_x, over_y, cov_x, cov_y, idx)
    return (featp.reshape(_B, 136, _N), px.reshape(_B, _N), py.reshape(_B, _N))


def _mlp_body(feat_ref, w1_ref, b1_ref, w2_ref, b2_ref, out_ref):
    feat = feat_ref[0]
    w1 = w1_ref[...]
    h = lax.dot_general(w1, feat, (((0,), (0,)), ((), ())),
                        preferred_element_type=jnp.float32)
    h = jnp.maximum(h + b1_ref[...].reshape(256, 1), 0.0)
    w2 = w2_ref[...]
    r = lax.dot_general(w2, h, (((0,), (0,)), ((), ())),
                        preferred_element_type=jnp.float32)
    out_ref[0] = r + b2_ref[...].reshape(8, 1)


def _mlp(featp, W1, b1, W2, b2):
    B, Cp, N = featp.shape
    W1p = jnp.pad(W1, ((0, Cp - W1.shape[0]), (0, 0)))
    W2p = jnp.pad(W2, ((0, 0), (0, 1)))
    b2p = jnp.pad(b2, (0, 1))
    out = pl.pallas_call(
        _mlp_body,
        grid=(B,),
        in_specs=[
            pl.BlockSpec((1, Cp, N), lambda b: (b, 0, 0)),
            pl.BlockSpec((Cp, 256), lambda b: (0, 0)),
            pl.BlockSpec((256,), lambda b: (0,)),
            pl.BlockSpec((256, 8), lambda b: (0, 0)),
            pl.BlockSpec((8,), lambda b: (0,)),
        ],
        out_specs=pl.BlockSpec((1, 8, N), lambda b: (b, 0, 0)),
        out_shape=jax.ShapeDtypeStruct((B, 8, N), jnp.float32),
    )(featp, W1p, b1, W2p, b2p)
    return out[:, :7, :]


def kernel(fine, coarse, W1, b1, W2, b2):
    B = fine.shape[0]
    pkey = jax.random.key(42)
    k1, k2 = jax.random.split(pkey)
    over = jax.random.uniform(k1, (B, _NOVER, 2), dtype=coarse.dtype)
    coverage = jax.random.uniform(k2, (B, _N - _NB, 2), dtype=coarse.dtype)

    over_x = over[..., 0] + 0.0
    over_y = over[..., 1] + 0.0
    cov_x = coverage[..., 0] + 0.0
    cov_y = coverage[..., 1] + 0.0
    over_x = over_x.reshape(-1)
    over_y = over_y.reshape(-1)
    cov_x = cov_x.reshape(-1)
    cov_y = cov_y.reshape(-1)
    coarse_flat = coarse.reshape(_B * _CH_C * _HC, _HC)
    fine_flat = fine.reshape(_B * _CH_F * _HF, _HF)

    unc = _sc_unc(coarse_flat, over_x, over_y)
    _, idx = jax.lax.top_k(unc, _NB)
    idx = idx.reshape(-1)

    featp, px, py = _sc_gather(fine_flat, coarse_flat, over_x, over_y,
                               cov_x, cov_y, idx)
    rend = _mlp(featp, W1, b1, W2, b2)
    points = jnp.stack([px, py], axis=-1)
    return rend, points
